# Initial kernel scaffold; baseline (speedup 1.0000x reference)
import jax
import jax.numpy as jnp
from jax.experimental import pallas as pl


def kernel(points):
    raise NotImplementedError("write your pallas kernel here")


# speedup vs baseline: 20.3015x; 20.3015x over previous
import jax
import jax.numpy as jnp
from jax import lax
from jax.experimental import pallas as pl
from jax.experimental.pallas import tpu as pltpu
from jax.experimental.pallas import tpu_sc as plsc

_B = 8
_N = 16384
_C = 6
_K = 1024

_L = 16
_G = 4
_NP = _N // _G
_NV = _NP // _L
_KT = _K // _G
_CHUNK = 2048
_UN = 4
_I32MAX = 2147483647


def _fps_body(flat_hbm, out_hbm, stage_v, c0_v, c1_v, c2_v, c3_v, c4_v, c5_v,
              dist_v, idx_v, cand_v, red_v, obuf_v, shared):
    cid = lax.axis_index("c")
    sid = lax.axis_index("s")
    b = cid * (_B // 2) + sid // _G
    r = sid % _G
    g = sid // _G
    qbase = r * _NP
    lane = lax.iota(jnp.int32, _L)
    chans = (c0_v, c1_v, c2_v, c3_v, c4_v, c5_v)

    def stage_chunk(ch, _):
        pltpu.sync_copy(flat_hbm.at[b, pl.ds(ch * _CHUNK * _C, _CHUNK * _C)],
                        stage_v)

        def tr(v, _):
            q6 = (jnp.full((_L,), v * _L, jnp.int32) + lane) * _C
            dst = ch * _CHUNK + v * _L
            for c in range(_C):
                chans[c][pl.ds(dst, _L)] = plsc.load_gather(
                    stage_v, [q6 + c] if c else [q6])
            return 0

        lax.fori_loop(0, _CHUNK // _L, tr, 0)
        return 0

    lax.fori_loop(0, _N // _CHUNK, stage_chunk, 0)

    inf16 = jnp.full((_L,), jnp.inf, jnp.float32)

    def init_d(v, _):
        dist_v[pl.ds(v * _L, _L)] = inf16
        return 0

    lax.fori_loop(0, _NV, init_d, 0)

    def it_body(i, far):
        fvec = jnp.full((_L,), far, jnp.int32)
        cx = plsc.load_gather(c0_v, [fvec])
        cy = plsc.load_gather(c1_v, [fvec])
        cz = plsc.load_gather(c2_v, [fvec])
        plsc.store_scatter(idx_v, [jnp.full((_L,), i, jnp.int32)], fvec,
                           mask=lane == 0)

        vmax0 = jnp.full((_L,), -jnp.inf, jnp.float32)
        varg0 = jnp.zeros((_L,), jnp.int32)

        @plsc.parallel_loop(0, _NV, step=1, unroll=_UN,
                            carry=(vmax0, varg0))
        def vloop(v, c2):
            vmax, varg = c2
            o = v * _L
            xs = c0_v[pl.ds(qbase + o, _L)]
            ys = c1_v[pl.ds(qbase + o, _L)]
            zs = c2_v[pl.ds(qbase + o, _L)]
            dx = xs - cx
            dy = ys - cy
            dz = zs - cz
            d = (dx * dx + dz * dz) + dy * dy
            nd = jnp.minimum(dist_v[pl.ds(o, _L)], d)
            dist_v[pl.ds(o, _L)] = nd
            vv = jnp.full((_L,), v, jnp.int32)
            take = (nd > vmax) | ((nd == vmax) & (vv < varg))
            vmax = jnp.where(take, nd, vmax)
            varg = jnp.where(take, vv, varg)
            return vmax, varg

        vmax, varg = vloop

        aidx = jnp.full((_L,), qbase, jnp.int32) + varg * _L + lane
        m = jnp.max(vmax)
        mm = jnp.full((_L,), m, jnp.float32)
        a = jnp.min(jnp.where(vmax == mm, aidx,
                              jnp.full((_L,), _I32MAX, jnp.int32)))
        cand_v[pl.ds(0, _L)] = plsc.bitcast(mm, jnp.int32)
        cand_v[pl.ds(_L, _L)] = jnp.full((_L,), a, jnp.int32)

        par = i % 2
        pltpu.sync_copy(cand_v,
                        shared.at[pl.ds(par * 512 + sid * 32, 2 * _L)])
        plsc.subcore_barrier()
        pltpu.sync_copy(shared.at[pl.ds(par * 512 + g * 128, 8 * _L)], red_v)

        bval = plsc.bitcast(red_v[pl.ds(0, _L)], jnp.float32)
        barg = red_v[pl.ds(_L, _L)]
        for t in range(1, _G):
            tv = plsc.bitcast(red_v[pl.ds(t * 32, _L)], jnp.float32)
            ta = red_v[pl.ds(t * 32 + _L, _L)]
            take = (tv > bval) | ((tv == bval) & (ta < barg))
            bval = jnp.where(take, tv, bval)
            barg = jnp.where(take, ta, barg)
        return jnp.max(barg)

    lax.fori_loop(0, _K, it_body, jnp.zeros((), jnp.int32))

    kbase = r * _KT

    def gat(w, _):
        iv = idx_v[pl.ds(kbase + w * _L, _L)]
        dst = (jnp.full((_L,), w * _L, jnp.int32) + lane) * _C
        for c in range(_C):
            plsc.store_scatter(obuf_v, [dst + c] if c else [dst],
                               plsc.load_gather(chans[c], [iv]))
        return 0

    lax.fori_loop(0, _KT // _L, gat, 0)
    pltpu.sync_copy(obuf_v, out_hbm.at[b, pl.ds(kbase * _C, _KT * _C)])


def _build(interpret=False):
    mesh = plsc.VectorSubcoreMesh(core_axis_name="c", subcore_axis_name="s",
                                  num_cores=2, num_subcores=16)
    return pl.kernel(
        _fps_body,
        out_type=jax.ShapeDtypeStruct((_B, _K * _C), jnp.float32),
        mesh=mesh,
        interpret=interpret,
        compiler_params=pltpu.CompilerParams(needs_layout_passes=False),
        scratch_types=(
            [pltpu.VMEM((_CHUNK * _C,), jnp.float32)]
            + [pltpu.VMEM((_N,), jnp.float32)] * _C
            + [
                pltpu.VMEM((_NP,), jnp.float32),
                pltpu.VMEM((_K,), jnp.int32),
                pltpu.VMEM((2 * _L,), jnp.int32),
                pltpu.VMEM((8 * _L,), jnp.int32),
                pltpu.VMEM((_KT * _C,), jnp.float32),
                pltpu.VMEM_SHARED((1024,), jnp.int32),
            ]
        ),
    )


def kernel(points):
    flat = points.reshape(_B, _N * _C)
    return _build()(flat).reshape(_B, _K, _C)
